# baseline (speedup 1.0000x reference)
import functools

import jax
import jax.numpy as jnp
from jax import lax
from jax.experimental import pallas as pl
from jax.experimental.pallas import tpu as pltpu
from jax.experimental.pallas import tpu_sc as plsc

_B, _T, _D = 4, 2048, 768
_ROWS = _B * _T
_NW = 32
_RPW = _ROWS // _NW
_CH = 32
_NCH = _RPW // _CH
_LANES = 16
_DC = _D // _LANES
_NCG = 4
_CGW = _DC // _NCG
_EPS = 1e-12


def _rsqrt(v):
    i = lax.bitcast_convert_type(v, jnp.int32)
    i = jnp.int32(0x5F3759DF) - lax.shift_right_logical(i, 1)
    y = lax.bitcast_convert_type(i, jnp.float32)
    for _ in range(3):
        y = y * (1.5 - 0.5 * v * y * y)
    return y


def _sc_body(x_hbm, seg_hbm, tok_hbm, pos_hbm, segemb_hbm, gamma_hbm,
             beta_hbm, out_hbm, idx_v, sid_v, segt_v, gam_v, bet_v,
             tok_v, pos_v, svs_v, svq_v, ab_v, bb_v, gsem0, gsem1,
             wsem0, wsem1, psem0, psem1):
    cid = lax.axis_index("c")
    sid = lax.axis_index("s")
    wid = sid * 2 + cid
    base = wid * _RPW
    gsems = (gsem0, gsem1)
    wsems = (wsem0, wsem1)
    psems = (psem0, psem1)

    pltpu.sync_copy(segemb_hbm, segt_v)
    pltpu.sync_copy(gamma_hbm, gam_v)
    pltpu.sync_copy(beta_hbm, bet_v)

    zeros = jnp.zeros((_LANES,), jnp.float32)
    iota = lax.iota(jnp.int32, _LANES)

    def fetch(c, buf):
        rbase = base + c * _CH
        t0 = rbase % _T
        pltpu.sync_copy(x_hbm.at[pl.ds(rbase, _CH)], idx_v.at[buf])
        pltpu.sync_copy(seg_hbm.at[pl.ds(rbase, _CH)],
                        sid_v.at[buf, pl.ds(0, _CH)])
        pltpu.async_copy(pos_hbm.at[pl.ds(t0, _CH)], pos_v.at[buf],
                         psems[buf])
        pltpu.async_copy(tok_hbm.at[idx_v.at[buf]], tok_v.at[buf],
                         gsems[buf])

    def fetch_wait(buf):
        pltpu.make_async_copy(pos_hbm.at[pl.ds(0, _CH)], pos_v.at[buf],
                              psems[buf]).wait()
        pltpu.make_async_copy(tok_hbm.at[idx_v.at[buf]], tok_v.at[buf],
                              gsems[buf]).wait()

    def wb_desc(c, buf):
        rbase = base + c * _CH
        return pltpu.make_async_copy(tok_v.at[buf],
                                     out_hbm.at[pl.ds(rbase, _CH)],
                                     wsems[buf])

    def compute(b):
        def arow_body(r):
            s_id = sid_v[b, pl.ds(r, _LANES)][0]
            acc = [zeros, zeros, zeros, zeros]
            for j in range(_DC):
                off = pl.ds(j * _LANES, _LANES)
                h = tok_v[b, r, off] + pos_v[b, r, off] + segt_v[s_id, off]
                tok_v[b, r, off] = h
                acc[j % 2] = acc[j % 2] + h
                acc[2 + j % 2] = acc[2 + j % 2] + h * h
            svs_v[r] = acc[0] + acc[1]
            svq_v[r] = acc[2] + acc[3]

        plsc.parallel_loop(0, _CH)(arow_body)

        for g in range(_CH // _LANES):
            rows = g * _LANES + iota
            ts = [zeros, zeros]
            tq = [zeros, zeros]
            for l in range(_LANES):
                col = jnp.full((_LANES,), l, jnp.int32)
                ts[l % 2] = ts[l % 2] + plsc.load_gather(svs_v, [rows, col])
                tq[l % 2] = tq[l % 2] + plsc.load_gather(svq_v, [rows, col])
            mu = (ts[0] + ts[1]) * (1.0 / _D)
            var = (tq[0] + tq[1]) * (1.0 / _D) - mu * mu
            y = _rsqrt(var + _EPS)
            ab_v[pl.ds(g * _LANES, _LANES)] = y
            bb_v[pl.ds(g * _LANES, _LANES)] = -mu * y

        for cg in range(_NCG):
            gs = [gam_v[pl.ds((cg * _CGW + j) * _LANES, _LANES)]
                  for j in range(_CGW)]
            bs = [bet_v[pl.ds((cg * _CGW + j) * _LANES, _LANES)]
                  for j in range(_CGW)]

            def nrow_body(r, gs=gs, bs=bs, cg=cg):
                a = ab_v[pl.ds(r, _LANES)][0]
                bb = bb_v[pl.ds(r, _LANES)][0]
                for j in range(_CGW):
                    off = pl.ds((cg * _CGW + j) * _LANES, _LANES)
                    h = tok_v[b, r, off]
                    tok_v[b, r, off] = (h * a + bb) * gs[j] + bs[j]

            plsc.parallel_loop(0, _CH, unroll=2)(nrow_body)

    fetch(0, 0)

    def chunk_pair(i, carry):
        c0 = 2 * i
        @pl.when(i > 0)
        def _wb1():
            wb_desc(c0 - 1, 1).wait()
        fetch(c0 + 1, 1)

        fetch_wait(0)
        compute(0)
        wb_desc(c0, 0).start()

        @pl.when(i < _NCH // 2 - 1)
        def _pf0():
            wb_desc(c0, 0).wait()
            fetch(c0 + 2, 0)

        fetch_wait(1)
        compute(1)
        wb_desc(c0 + 1, 1).start()
        return carry

    lax.fori_loop(0, _NCH // 2, chunk_pair, 0)
    wb_desc(_NCH - 2, 0).wait()
    wb_desc(_NCH - 1, 1).wait()


@jax.jit
def _emb_ln(xf, sf, tok_emb, pos_emb, seg_emb, gamma, beta):
    mesh = plsc.VectorSubcoreMesh(core_axis_name="c", subcore_axis_name="s")
    call = functools.partial(
        pl.kernel,
        mesh=mesh,
        out_type=jax.ShapeDtypeStruct((_ROWS, _D), jnp.float32),
        compiler_params=pltpu.CompilerParams(needs_layout_passes=False),
        scratch_types=[
            pltpu.VMEM((2, _CH), jnp.int32),
            pltpu.VMEM((2, _CH + _LANES), jnp.int32),
            pltpu.VMEM((2, _D), jnp.float32),
            pltpu.VMEM((_D,), jnp.float32),
            pltpu.VMEM((_D,), jnp.float32),
            pltpu.VMEM((2, _CH, _D), jnp.float32),
            pltpu.VMEM((2, _CH, _D), jnp.float32),
            pltpu.VMEM((_CH, _LANES), jnp.float32),
            pltpu.VMEM((_CH, _LANES), jnp.float32),
            pltpu.VMEM((_CH + _LANES,), jnp.float32),
            pltpu.VMEM((_CH + _LANES,), jnp.float32),
            pltpu.SemaphoreType.DMA,
            pltpu.SemaphoreType.DMA,
            pltpu.SemaphoreType.DMA,
            pltpu.SemaphoreType.DMA,
            pltpu.SemaphoreType.DMA,
            pltpu.SemaphoreType.DMA,
        ],
    )(_sc_body)
    return call(xf, sf, tok_emb, pos_emb, seg_emb, gamma, beta)


def kernel(x, segments, tok_emb, pos_emb, seg_emb, gamma, beta):
    xf = x.reshape(-1)
    sf = segments.reshape(-1)
    out = _emb_ln(xf, sf, tok_emb, pos_emb, seg_emb, gamma, beta)
    return out.reshape(_B, _T, _D)


# speedup vs baseline: 1.3592x; 1.3592x over previous
import functools

import jax
import jax.numpy as jnp
from jax import lax
from jax.experimental import pallas as pl
from jax.experimental.pallas import tpu as pltpu
from jax.experimental.pallas import tpu_sc as plsc

_B, _T, _D = 4, 2048, 768
_ROWS = _B * _T
_NW = 32
_RPW = _ROWS // _NW
_CH = 32
_NCH = _RPW // _CH
_LANES = 16
_DC = _D // _LANES
_NCG = 6
_CGW = _DC // _NCG
_EPS = 1e-12


def _rsqrt(v):
    i = lax.bitcast_convert_type(v, jnp.int32)
    i = jnp.int32(0x5F3759DF) - lax.shift_right_logical(i, 1)
    y = lax.bitcast_convert_type(i, jnp.float32)
    for _ in range(3):
        y = y * (1.5 - 0.5 * v * y * y)
    return y


def _sc_body(x_hbm, seg_hbm, tok_hbm, pos_hbm, segemb_hbm, gamma_hbm,
             beta_hbm, out_hbm, idx_v, sid_v, segt_v, gam_v, bet_v,
             tok_v, pos_v, svs_v, svq_v, ab_v, bb_v, gsem0, gsem1,
             wsem0, wsem1, psem0, psem1):
    cid = lax.axis_index("c")
    sid = lax.axis_index("s")
    wid = sid * 2 + cid
    base = wid * _RPW
    gsems = (gsem0, gsem1)
    wsems = (wsem0, wsem1)
    psems = (psem0, psem1)

    pltpu.sync_copy(segemb_hbm, segt_v)
    pltpu.sync_copy(gamma_hbm, gam_v)
    pltpu.sync_copy(beta_hbm, bet_v)

    zeros = jnp.zeros((_LANES,), jnp.float32)
    iota = lax.iota(jnp.int32, _LANES)

    def fetch(c, buf):
        rbase = base + c * _CH
        t0 = rbase % _T
        pltpu.sync_copy(x_hbm.at[pl.ds(rbase, _CH)], idx_v.at[buf])
        pltpu.sync_copy(seg_hbm.at[pl.ds(rbase, _CH)],
                        sid_v.at[buf, pl.ds(0, _CH)])
        pltpu.async_copy(pos_hbm.at[pl.ds(t0, _CH)], pos_v.at[buf],
                         psems[buf])
        pltpu.async_copy(tok_hbm.at[idx_v.at[buf]], tok_v.at[buf],
                         gsems[buf])

    def fetch_wait(buf):
        pltpu.make_async_copy(pos_hbm.at[pl.ds(0, _CH)], pos_v.at[buf],
                              psems[buf]).wait()
        pltpu.make_async_copy(tok_hbm.at[idx_v.at[buf]], tok_v.at[buf],
                              gsems[buf]).wait()

    def wb_desc(c, buf):
        rbase = base + c * _CH
        return pltpu.make_async_copy(tok_v.at[buf],
                                     out_hbm.at[pl.ds(rbase, _CH)],
                                     wsems[buf])

    def compute(b):
        for cg in range(_NCG):
            sg0 = [segt_v[0, pl.ds((cg * _CGW + j) * _LANES, _LANES)]
                   for j in range(_CGW)]
            sg1 = [segt_v[1, pl.ds((cg * _CGW + j) * _LANES, _LANES)]
                   for j in range(_CGW)]

            def arow_body(r, cg=cg, sg0=sg0, sg1=sg1):
                s_id = sid_v[b, pl.ds(r, _LANES)][0]
                p = s_id != 0
                acc = [zeros, zeros, zeros, zeros]
                for j in range(_CGW):
                    off = pl.ds((cg * _CGW + j) * _LANES, _LANES)
                    segc = jnp.where(p, sg1[j], sg0[j])
                    h = tok_v[b, r, off] + pos_v[b, r, off] + segc
                    tok_v[b, r, off] = h
                    acc[j % 2] = acc[j % 2] + h
                    acc[2 + j % 2] = acc[2 + j % 2] + h * h
                if cg == 0:
                    svs_v[r] = acc[0] + acc[1]
                    svq_v[r] = acc[2] + acc[3]
                else:
                    svs_v[r] = svs_v[r] + (acc[0] + acc[1])
                    svq_v[r] = svq_v[r] + (acc[2] + acc[3])

            plsc.parallel_loop(0, _CH, unroll=2)(arow_body)

        for g in range(_CH // _LANES):
            rows = g * _LANES + iota
            ts = [zeros, zeros]
            tq = [zeros, zeros]
            for l in range(_LANES):
                col = jnp.full((_LANES,), l, jnp.int32)
                ts[l % 2] = ts[l % 2] + plsc.load_gather(svs_v, [rows, col])
                tq[l % 2] = tq[l % 2] + plsc.load_gather(svq_v, [rows, col])
            mu = (ts[0] + ts[1]) * (1.0 / _D)
            var = (tq[0] + tq[1]) * (1.0 / _D) - mu * mu
            y = _rsqrt(var + _EPS)
            ab_v[pl.ds(g * _LANES, _LANES)] = y
            bb_v[pl.ds(g * _LANES, _LANES)] = -mu * y

        for cg in range(_NCG):
            gs = [gam_v[pl.ds((cg * _CGW + j) * _LANES, _LANES)]
                  for j in range(_CGW)]
            bs = [bet_v[pl.ds((cg * _CGW + j) * _LANES, _LANES)]
                  for j in range(_CGW)]

            def nrow_body(r, gs=gs, bs=bs, cg=cg):
                a = ab_v[pl.ds(r, _LANES)][0]
                bb = bb_v[pl.ds(r, _LANES)][0]
                for j in range(_CGW):
                    off = pl.ds((cg * _CGW + j) * _LANES, _LANES)
                    h = tok_v[b, r, off]
                    tok_v[b, r, off] = (h * a + bb) * gs[j] + bs[j]

            plsc.parallel_loop(0, _CH, unroll=2)(nrow_body)

    fetch(0, 0)

    def chunk_pair(i, carry):
        c0 = 2 * i
        @pl.when(i > 0)
        def _wb1():
            wb_desc(c0 - 1, 1).wait()
        fetch(c0 + 1, 1)

        fetch_wait(0)
        compute(0)
        wb_desc(c0, 0).start()

        @pl.when(i < _NCH // 2 - 1)
        def _pf0():
            wb_desc(c0, 0).wait()
            fetch(c0 + 2, 0)

        fetch_wait(1)
        compute(1)
        wb_desc(c0 + 1, 1).start()
        return carry

    lax.fori_loop(0, _NCH // 2, chunk_pair, 0)
    wb_desc(_NCH - 2, 0).wait()
    wb_desc(_NCH - 1, 1).wait()


@jax.jit
def _emb_ln(xf, sf, tok_emb, pos_emb, seg_emb, gamma, beta):
    mesh = plsc.VectorSubcoreMesh(core_axis_name="c", subcore_axis_name="s")
    call = functools.partial(
        pl.kernel,
        mesh=mesh,
        out_type=jax.ShapeDtypeStruct((_ROWS, _D), jnp.float32),
        compiler_params=pltpu.CompilerParams(needs_layout_passes=False),
        scratch_types=[
            pltpu.VMEM((2, _CH), jnp.int32),
            pltpu.VMEM((2, _CH + _LANES), jnp.int32),
            pltpu.VMEM((2, _D), jnp.float32),
            pltpu.VMEM((_D,), jnp.float32),
            pltpu.VMEM((_D,), jnp.float32),
            pltpu.VMEM((2, _CH, _D), jnp.float32),
            pltpu.VMEM((2, _CH, _D), jnp.float32),
            pltpu.VMEM((_CH, _LANES), jnp.float32),
            pltpu.VMEM((_CH, _LANES), jnp.float32),
            pltpu.VMEM((_CH + _LANES,), jnp.float32),
            pltpu.VMEM((_CH + _LANES,), jnp.float32),
            pltpu.SemaphoreType.DMA,
            pltpu.SemaphoreType.DMA,
            pltpu.SemaphoreType.DMA,
            pltpu.SemaphoreType.DMA,
            pltpu.SemaphoreType.DMA,
            pltpu.SemaphoreType.DMA,
        ],
    )(_sc_body)
    return call(xf, sf, tok_emb, pos_emb, seg_emb, gamma, beta)


def kernel(x, segments, tok_emb, pos_emb, seg_emb, gamma, beta):
    xf = x.reshape(-1)
    sf = segments.reshape(-1)
    out = _emb_ln(xf, sf, tok_emb, pos_emb, seg_emb, gamma, beta)
    return out.reshape(_B, _T, _D)
